# baseline (speedup 1.0000x reference)
import functools

import jax
import jax.numpy as jnp
from jax import lax
from jax.experimental import pallas as pl
from jax.experimental.pallas import tpu as pltpu
from jax.experimental.pallas import tpu_sc as plsc

NC = 2
NS = 16
LANES = 16
NBUF = 4


def _make_emb(B, L, V, D):
  NW = NC * NS
  BPW = B // NW
  mesh = plsc.VectorSubcoreMesh(
      core_axis_name="c", subcore_axis_name="s",
      num_cores=NC, num_subcores=NS)

  scratch = [
      pltpu.VMEM((L, BPW), jnp.int32),
      pltpu.VMEM((L, D), jnp.float32),
      pltpu.VMEM((NBUF, BPW, D), jnp.float32),
  ] + [pltpu.SemaphoreType.DMA] * (2 * NBUF)

  @functools.partial(
      pl.kernel, mesh=mesh,
      out_type=jax.ShapeDtypeStruct((B, L, D), jnp.float32),
      scratch_types=scratch,
      compiler_params=pltpu.CompilerParams(use_tc_tiling_on_sc=False))
  def emb(ids_hbm, word_hbm, pos_hbm, out_hbm, idx_v, pos_v, rows_v, *sems):
    gsem = sems[:NBUF]
    wsem = sems[NBUF:]
    wid = lax.axis_index("s") * NC + lax.axis_index("c")
    b0 = wid * BPW

    pltpu.sync_copy(ids_hbm.at[wid], idx_v)
    pltpu.sync_copy(pos_hbm, pos_v)

    def start_gather(l, b):
      pltpu.async_copy(word_hbm.at[idx_v.at[l]], rows_v.at[b], gsem[b])

    def wait_gather(b):
      pltpu.make_async_copy(
          word_hbm.at[pl.ds(0, BPW)], rows_v.at[b], gsem[b]).wait()

    def start_write(l, b):
      pltpu.async_copy(rows_v.at[b], out_hbm.at[pl.ds(b0, BPW), l], wsem[b])

    def wait_write(l, b):
      pltpu.make_async_copy(
          rows_v.at[b], out_hbm.at[pl.ds(b0, BPW), l], wsem[b]).wait()

    for b in range(NBUF - 1):
      start_gather(b, b)

    def outer(i, _):
      l0 = i * NBUF
      for b in range(NBUF):
        l = l0 + b
        wait_gather(b)
        pv = [pos_v[l, pl.ds(LANES * j, LANES)] for j in range(D // LANES)]

        def radd(r, _):
          for j in range(D // LANES):
            sl = pl.ds(LANES * j, LANES)
            rows_v[b, r, sl] = rows_v[b, r, sl] + pv[j]
          return 0

        lax.fori_loop(0, BPW, radd, 0)
        start_write(l, b)
        ln = l + NBUF - 1
        bn = (b + NBUF - 1) % NBUF

        @pl.when(ln < L)
        def _():
          @pl.when(ln >= NBUF)
          def _():
            wait_write(ln - NBUF, bn)
          start_gather(ln, bn)
      return 0

    lax.fori_loop(0, L // NBUF, outer, 0)

    for b in range(NBUF):
      wait_write(L - NBUF + b, b)

  return emb


def kernel(input_ids, word_table, pos_table):
  B, L = input_ids.shape
  V, D = word_table.shape
  NW = NC * NS
  BPW = B // NW
  ids_prep = input_ids.astype(jnp.int32).T.reshape(L, NW, BPW).transpose(1, 0, 2)
  emb = _make_emb(B, L, V, D)
  return emb(ids_prep, word_table, pos_table[:L])
